# Initial kernel scaffold; baseline (speedup 1.0000x reference)
import functools

import jax
import jax.numpy as jnp
from jax import lax
from jax.experimental import pallas as pl
from jax.experimental.pallas import tpu as pltpu
from jax.experimental.pallas import tpu_sc as plsc

N = 10000
E = 320000
D = 128
NT = 5000
C = 100
EPS = 1e-5

NC = 2
NS = 16
NW = NC * NS

CHUNK = 128
NCH = -(-(E // NW) // CHUNK)
EP = NW * NCH * CHUNK
ZROWS = 626
NP = NS * ZROWS

TROWS = 160
NTP = NW * TROWS

_SC_MESH = plsc.VectorSubcoreMesh(core_axis_name="c", subcore_axis_name="s")


def _make_agg(with_deg: bool):
    out_type = [jax.ShapeDtypeStruct((NC, NP, D), jnp.float32)]
    scratch = [
        pltpu.VMEM_SHARED((NP, D), jnp.float32),
        pltpu.VMEM((NCH, CHUNK), jnp.int32),
        pltpu.VMEM((NCH, CHUNK), jnp.int32),
        pltpu.VMEM((CHUNK, D), jnp.float32),
        pltpu.VMEM((CHUNK, D), jnp.float32),
        pltpu.SemaphoreType.DMA,
        pltpu.SemaphoreType.DMA,
    ]
    if with_deg:
        out_type.append(jax.ShapeDtypeStruct((NC, NP, 16), jnp.float32))
        scratch += [
            pltpu.VMEM_SHARED((NP, 16), jnp.float32),
            pltpu.VMEM((CHUNK, 16), jnp.float32),
        ]

    def body(*refs):
        if with_deg:
            (feat, srcp, dstp, znd, zdg, ones_h,
             sum_out, deg_out,
             acc, srcv, dstv, bufA, bufB, semA, semB, dacc, onesv) = refs
        else:
            (feat, srcp, dstp, znd,
             sum_out,
             acc, srcv, dstv, bufA, bufB, semA, semB) = refs

        c = lax.axis_index("c")
        s = lax.axis_index("s")
        w = c * NS + s
        z0 = s * ZROWS

        pltpu.sync_copy(znd.at[pl.ds(z0, ZROWS)], acc.at[pl.ds(z0, ZROWS)])
        if with_deg:
            pltpu.sync_copy(zdg.at[pl.ds(z0, ZROWS)], dacc.at[pl.ds(z0, ZROWS)])
            pltpu.sync_copy(ones_h, onesv)
        pltpu.sync_copy(srcp.at[w], srcv)
        pltpu.sync_copy(dstp.at[w], dstv)
        plsc.subcore_barrier()

        def scatter(buf, j):
            pltpu.sync_copy(buf, acc.at[dstv.at[j]], add=True)
            if with_deg:
                pltpu.sync_copy(onesv, dacc.at[dstv.at[j]], add=True)

        def wait(buf, sem):
            pltpu.make_async_copy(feat.at[pl.ds(0, CHUNK)], buf, sem).wait()

        pltpu.async_copy(feat.at[srcv.at[0]], bufA, semA)

        def step(i, carry):
            j0 = i * 2
            pltpu.async_copy(feat.at[srcv.at[j0 + 1]], bufB, semB)
            wait(bufA, semA)
            scatter(bufA, j0)
            pltpu.async_copy(feat.at[srcv.at[j0 + 2]], bufA, semA)
            wait(bufB, semB)
            scatter(bufB, j0 + 1)
            return carry

        lax.fori_loop(0, (NCH - 1) // 2, step, 0)
        wait(bufA, semA)
        scatter(bufA, NCH - 1)

        plsc.subcore_barrier()
        pltpu.sync_copy(acc.at[pl.ds(z0, ZROWS)], sum_out.at[c, pl.ds(z0, ZROWS)])
        if with_deg:
            pltpu.sync_copy(dacc.at[pl.ds(z0, ZROWS)], deg_out.at[c, pl.ds(z0, ZROWS)])

    return pl.kernel(
        body,
        out_type=tuple(out_type) if with_deg else out_type[0],
        mesh=_SC_MESH,
        scratch_types=scratch,
        name="sage_agg_deg" if with_deg else "sage_agg",
    )


_agg_deg = _make_agg(True)
_agg = _make_agg(False)


def _gather_body(feat, idxp, out, idxv, buf, sem):
    c = lax.axis_index("c")
    s = lax.axis_index("s")
    w = c * NS + s
    pltpu.sync_copy(idxp.at[w], idxv)
    for t in range(2):
        pltpu.async_copy(feat.at[idxv.at[t]], buf, sem).wait()
        pltpu.sync_copy(buf, out.at[pl.ds(w * TROWS + t * 80, 80)])


_gather = pl.kernel(
    _gather_body,
    out_type=jax.ShapeDtypeStruct((NTP, D), jnp.float32),
    mesh=_SC_MESH,
    scratch_types=[
        pltpu.VMEM((2, 80), jnp.int32),
        pltpu.VMEM((80, D), jnp.float32),
        pltpu.SemaphoreType.DMA,
    ],
    name="sage_gather_xt",
)


def _layer_body(sums_ref, deg_ref, x_ref, wl_ref, bl_ref, wr_ref, g_ref, b_ref, o_ref):
    su = sums_ref[...]
    dg = deg_ref[...]
    s = su[0, :N, :] + su[1, :N, :]
    deg = dg[0, :N, 0:1] + dg[1, :N, 0:1]
    agg = s * (1.0 / jnp.maximum(deg, 1.0))
    x = x_ref[...]
    hp = jax.lax.dot(agg, wl_ref[...], precision=lax.Precision.HIGHEST,
                     preferred_element_type=jnp.float32)
    hp = hp + jax.lax.dot(x, wr_ref[...], precision=lax.Precision.HIGHEST,
                          preferred_element_type=jnp.float32)
    hp = hp + bl_ref[0][None, :]
    scale = g_ref[0] * jax.lax.rsqrt(jnp.float32(1.0 + EPS))
    h = hp * scale[None, :] + b_ref[0][None, :]
    o_ref[...] = jnp.maximum(h, 0.0)


_layer = pl.pallas_call(
    _layer_body,
    out_shape=jax.ShapeDtypeStruct((N, D), jnp.float32),
)


def _head_body(xt_ref, ci_ref, w_ref, b_ref, o_ref):
    xt = xt_ref[...][:NT, :]
    ci = ci_ref[...]
    colsum = jnp.sum(ci, axis=0)[:, None]
    cf = jax.lax.dot_general(ci, xt, (((0,), (0,)), ((), ())),
                             precision=lax.Precision.HIGHEST,
                             preferred_element_type=jnp.float32)
    cf = cf / colsum
    rowmax = jnp.max(ci, axis=1, keepdims=True)
    colidx = lax.broadcasted_iota(jnp.int32, (NT, C), 1)
    amax = jnp.min(jnp.where(ci == rowmax, colidx, C), axis=1, keepdims=True)
    oh = (colidx == amax).astype(jnp.float32)
    x1 = jax.lax.dot(oh, cf, precision=lax.Precision.HIGHEST,
                     preferred_element_type=jnp.float32)
    w = w_ref[...]
    wa = w[:D, :]
    wb = w[D:, :]
    b = b_ref[0][None, :]
    top = (jax.lax.dot(xt, wa, precision=lax.Precision.HIGHEST,
                       preferred_element_type=jnp.float32)
           + jax.lax.dot(x1, wb, precision=lax.Precision.HIGHEST,
                         preferred_element_type=jnp.float32) + b)
    bot = (jax.lax.dot(x1, wa, precision=lax.Precision.HIGHEST,
                       preferred_element_type=jnp.float32)
           + jax.lax.dot(xt, wb, precision=lax.Precision.HIGHEST,
                         preferred_element_type=jnp.float32) + b)
    o_ref[...] = jnp.concatenate([top, bot], axis=0)


_head = pl.pallas_call(
    _head_body,
    out_shape=jax.ShapeDtypeStruct((2 * NT, 4 * 64), jnp.float32),
)


def kernel(x, edge_index, cluster_index, train_mask, conv0_Wl, conv0_bl, conv0_Wr,
           conv1_Wl, conv1_bl, conv1_Wr, bn0_gamma, bn0_beta, bn1_gamma, bn1_beta,
           fc1_W, fc1_b):
    src = edge_index[0].astype(jnp.int32)
    dst = edge_index[1].astype(jnp.int32)
    pad = EP - E
    srcp = jnp.concatenate([src, jnp.zeros((pad,), jnp.int32)]).reshape(NW, NCH, CHUNK)
    dstp = jnp.concatenate([dst, jnp.full((pad,), N, jnp.int32)]).reshape(NW, NCH, CHUNK)
    znd = jnp.zeros((NP, D), jnp.float32)
    zdg = jnp.zeros((NP, 16), jnp.float32)
    ones_h = jnp.ones((CHUNK, 16), jnp.float32)

    sums0, deg16 = _agg_deg(x, srcp, dstp, znd, zdg, ones_h)
    h0 = _layer(sums0, deg16, x, conv0_Wl, conv0_bl.reshape(1, D), conv0_Wr,
                bn0_gamma.reshape(1, D), bn0_beta.reshape(1, D))
    sums1 = _agg(h0, srcp, dstp, znd)
    h1 = _layer(sums1, deg16, h0, conv1_Wl, conv1_bl.reshape(1, D), conv1_Wr,
                bn1_gamma.reshape(1, D), bn1_beta.reshape(1, D))

    tmp = jnp.concatenate([train_mask.astype(jnp.int32),
                           jnp.zeros((NTP - NT,), jnp.int32)]).reshape(NW, 2, 80)
    xt = _gather(h1, tmp)
    return _head(xt, cluster_index, fc1_W, fc1_b.reshape(1, 4 * 64))


# speedup vs baseline: 2.6695x; 2.6695x over previous
import jax
import jax.numpy as jnp
from jax import lax
from jax.experimental import pallas as pl
from jax.experimental.pallas import tpu as pltpu
from jax.experimental.pallas import tpu_sc as plsc

N = 10000
E = 320000
D = 128
NT = 5000
C = 100
EPS = 1e-5

NC = 2
NS = 16
NW = NC * NS

CHUNK = 64
GRP = 8
NGRP = 20
NCH = NGRP * GRP
EP = NW * NCH * CHUNK
ZROWS = 632
NP = NS * ZROWS

TROWS = 160
NTP = NW * TROWS

_SC_MESH = plsc.VectorSubcoreMesh(core_axis_name="c", subcore_axis_name="s")


def _make_agg(with_deg: bool):
    out_type = [jax.ShapeDtypeStruct((NC, NP, D), jnp.float32)]
    scratch = [
        pltpu.VMEM_SHARED((NP, D), jnp.float32),
        pltpu.VMEM((GRP, CHUNK), jnp.int32),
        pltpu.VMEM((GRP, CHUNK), jnp.int32),
        pltpu.VMEM((CHUNK, D), jnp.float32),
        pltpu.VMEM((CHUNK, D), jnp.float32),
        pltpu.SemaphoreType.DMA,
        pltpu.SemaphoreType.DMA,
        pltpu.SemaphoreType.DMA,
    ]

    def body(*refs):
        (feat, srcp, dstp, sum_out,
         acc, srcv, dstv, d0, d1, sem0, sem1, semI) = refs

        c = lax.axis_index("c")
        s = lax.axis_index("s")
        w = c * NS + s
        z0 = s * ZROWS
        bufs = (d0, d1)
        sems = (sem0, sem1)

        zvec = jnp.zeros((16,), jnp.float32)

        def zrow(r, carry):
            for k in range(D // 16):
                d0[r, pl.ds(k * 16, 16)] = zvec
            return carry

        lax.fori_loop(0, CHUNK, zrow, 0)
        for k in range(9):
            pltpu.sync_copy(d0, acc.at[pl.ds(z0 + CHUNK * k, CHUNK)])
        pltpu.sync_copy(d0.at[pl.ds(0, 56)], acc.at[pl.ds(z0 + 576, 56)])
        plsc.subcore_barrier()

        def wait_data(r):
            pltpu.make_async_copy(feat.at[srcv.at[0]], bufs[r], sems[r]).wait()

        def wait_idx(dst_v):
            pltpu.make_async_copy(srcp.at[0, pl.ds(0, GRP)], dst_v, semI).wait()

        def gstep(m, carry):
            pltpu.async_copy(srcp.at[w, pl.ds(m * GRP, GRP)], srcv, semI)
            pltpu.async_copy(dstp.at[w, pl.ds(m * GRP, GRP)], dstv, semI)
            wait_idx(srcv)
            wait_idx(dstv)
            pltpu.async_copy(feat.at[srcv.at[0]], d0, sem0)
            for r in range(GRP):
                if r + 1 < GRP:
                    pltpu.async_copy(feat.at[srcv.at[r + 1]], bufs[(r + 1) % 2],
                                     sems[(r + 1) % 2])
                wait_data(r % 2)
                pltpu.sync_copy(bufs[r % 2], acc.at[dstv.at[r]], add=True)
            return carry

        lax.fori_loop(0, NGRP, gstep, 0)

        plsc.subcore_barrier()
        for k in range(9):
            pltpu.sync_copy(acc.at[pl.ds(z0 + CHUNK * k, CHUNK)], d0)
            pltpu.sync_copy(d0, sum_out.at[c, pl.ds(z0 + CHUNK * k, CHUNK)])
        pltpu.sync_copy(acc.at[pl.ds(z0 + 576, 56)], d0.at[pl.ds(0, 56)])
        pltpu.sync_copy(d0.at[pl.ds(0, 56)], sum_out.at[c, pl.ds(z0 + 576, 56)])

    return pl.kernel(
        body,
        out_type=out_type[0],
        mesh=_SC_MESH,
        scratch_types=scratch,
        name="sage_agg",
    )


_agg = _make_agg(False)


def _gather_body(feat, idxp, out, idxv, buf, sem):
    c = lax.axis_index("c")
    s = lax.axis_index("s")
    w = c * NS + s
    pltpu.sync_copy(idxp.at[w], idxv)
    for t in range(2):
        pltpu.async_copy(feat.at[idxv.at[t]], buf, sem).wait()
        pltpu.sync_copy(buf, out.at[pl.ds(w * TROWS + t * 80, 80)])


_gather = pl.kernel(
    _gather_body,
    out_type=jax.ShapeDtypeStruct((NTP, D), jnp.float32),
    mesh=_SC_MESH,
    scratch_types=[
        pltpu.VMEM((2, 80), jnp.int32),
        pltpu.VMEM((80, D), jnp.float32),
        pltpu.SemaphoreType.DMA,
    ],
    name="sage_gather_xt",
)


RB = 1000


def _layer_body(sums_ref, deg_ref, x_ref, wl_ref, bl_ref, wr_ref, g_ref, b_ref, o_ref):
    su = sums_ref[...]
    deg = deg_ref[...]
    s = su[0] + su[1]
    agg = s * (1.0 / jnp.maximum(deg, 1.0))
    x = x_ref[...]
    hp = jax.lax.dot(agg, wl_ref[...], precision=lax.Precision.HIGHEST,
                     preferred_element_type=jnp.float32)
    hp = hp + jax.lax.dot(x, wr_ref[...], precision=lax.Precision.HIGHEST,
                          preferred_element_type=jnp.float32)
    hp = hp + bl_ref[0][None, :]
    scale = g_ref[0] * jax.lax.rsqrt(jnp.float32(1.0 + EPS))
    h = hp * scale[None, :] + b_ref[0][None, :]
    o_ref[...] = jnp.maximum(h, 0.0)


_layer = pl.pallas_call(
    _layer_body,
    grid=(N // RB,),
    in_specs=[
        pl.BlockSpec((NC, RB, D), lambda i: (0, i, 0)),
        pl.BlockSpec((RB, 1), lambda i: (i, 0)),
        pl.BlockSpec((RB, D), lambda i: (i, 0)),
        pl.BlockSpec((D, D), lambda i: (0, 0)),
        pl.BlockSpec((1, D), lambda i: (0, 0)),
        pl.BlockSpec((D, D), lambda i: (0, 0)),
        pl.BlockSpec((1, D), lambda i: (0, 0)),
        pl.BlockSpec((1, D), lambda i: (0, 0)),
    ],
    out_specs=pl.BlockSpec((RB, D), lambda i: (i, 0)),
    out_shape=jax.ShapeDtypeStruct((N, D), jnp.float32),
)


def _head_body(xt_ref, ci_ref, w_ref, b_ref, o_ref):
    xt = xt_ref[...][:NT, :]
    ci = ci_ref[...]
    colsum = jnp.sum(ci, axis=0)[:, None]
    cf = jax.lax.dot_general(ci, xt, (((0,), (0,)), ((), ())),
                             precision=lax.Precision.HIGHEST,
                             preferred_element_type=jnp.float32)
    cf = cf / colsum
    rowmax = jnp.max(ci, axis=1, keepdims=True)
    colidx = lax.broadcasted_iota(jnp.int32, (NT, C), 1)
    amax = jnp.min(jnp.where(ci == rowmax, colidx, C), axis=1, keepdims=True)
    oh = (colidx == amax).astype(jnp.float32)
    x1 = jax.lax.dot(oh, cf, precision=lax.Precision.HIGHEST,
                     preferred_element_type=jnp.float32)
    w = w_ref[...]
    wa = w[:D, :]
    wb = w[D:, :]
    b = b_ref[0][None, :]
    top = (jax.lax.dot(xt, wa, precision=lax.Precision.HIGHEST,
                       preferred_element_type=jnp.float32)
           + jax.lax.dot(x1, wb, precision=lax.Precision.HIGHEST,
                         preferred_element_type=jnp.float32) + b)
    bot = (jax.lax.dot(x1, wa, precision=lax.Precision.HIGHEST,
                       preferred_element_type=jnp.float32)
           + jax.lax.dot(xt, wb, precision=lax.Precision.HIGHEST,
                         preferred_element_type=jnp.float32) + b)
    o_ref[...] = jnp.concatenate([top, bot], axis=0)


_head = pl.pallas_call(
    _head_body,
    out_shape=jax.ShapeDtypeStruct((2 * NT, 4 * 64), jnp.float32),
)


def kernel(x, edge_index, cluster_index, train_mask, conv0_Wl, conv0_bl, conv0_Wr,
           conv1_Wl, conv1_bl, conv1_Wr, bn0_gamma, bn0_beta, bn1_gamma, bn1_beta,
           fc1_W, fc1_b):
    src = edge_index[0].astype(jnp.int32)
    dst = edge_index[1].astype(jnp.int32)
    pad = EP - E
    srcp = jnp.concatenate([src, jnp.zeros((pad,), jnp.int32)]).reshape(NW, NCH, CHUNK)
    dstp = jnp.concatenate([dst, jnp.full((pad,), N, jnp.int32)]).reshape(NW, NCH, CHUNK)

    sums0 = _agg(x, srcp, dstp)
    deg_col = jax.ops.segment_sum(jnp.ones((E,), jnp.float32), dst,
                                  num_segments=N).reshape(N, 1)
    h0 = _layer(sums0, deg_col, x, conv0_Wl, conv0_bl.reshape(1, D), conv0_Wr,
                bn0_gamma.reshape(1, D), bn0_beta.reshape(1, D))
    sums1 = _agg(h0, srcp, dstp)
    h1 = _layer(sums1, deg_col, h0, conv1_Wl, conv1_bl.reshape(1, D), conv1_Wr,
                bn1_gamma.reshape(1, D), bn1_beta.reshape(1, D))

    tmp = jnp.concatenate([train_mask.astype(jnp.int32),
                           jnp.zeros((NTP - NT,), jnp.int32)]).reshape(NW, 2, 80)
    xt = _gather(h1, tmp)
    return _head(xt, cluster_index, fc1_W, fc1_b.reshape(1, 4 * 64))


# speedup vs baseline: 2.6709x; 1.0005x over previous
import jax
import jax.numpy as jnp
from jax import lax
from jax.experimental import pallas as pl
from jax.experimental.pallas import tpu as pltpu
from jax.experimental.pallas import tpu_sc as plsc

N = 10000
E = 320000
D = 128
NT = 5000
C = 100
EPS = 1e-5

NC = 2
NS = 16
NW = NC * NS

CHUNK = 64
GRP = 8
NGRP = 20
NCH = NGRP * GRP
EP = NW * NCH * CHUNK
ZROWS = 632
NP = NS * ZROWS

TROWS = 160
NTP = NW * TROWS

_SC_MESH = plsc.VectorSubcoreMesh(core_axis_name="c", subcore_axis_name="s",
                                  num_cores=NC, num_subcores=NS)


def _make_agg(with_deg: bool):
    out_type = [jax.ShapeDtypeStruct((NC, NP, D), jnp.float32)]
    scratch = [
        pltpu.VMEM_SHARED((NP, D), jnp.float32),
        pltpu.VMEM((GRP, CHUNK), jnp.int32),
        pltpu.VMEM((GRP, CHUNK), jnp.int32),
        pltpu.VMEM((CHUNK, D), jnp.float32),
        pltpu.VMEM((CHUNK, D), jnp.float32),
        pltpu.SemaphoreType.DMA,
        pltpu.SemaphoreType.DMA,
        pltpu.SemaphoreType.DMA,
    ]

    def body(*refs):
        (feat, srcp, dstp, sum_out,
         acc, srcv, dstv, d0, d1, sem0, sem1, semI) = refs

        c = lax.axis_index("c")
        s = lax.axis_index("s")
        w = c * NS + s
        z0 = s * ZROWS
        bufs = (d0, d1)
        sems = (sem0, sem1)

        zvec = jnp.zeros((16,), jnp.float32)

        def zrow(r, carry):
            for k in range(D // 16):
                d0[r, pl.ds(k * 16, 16)] = zvec
            return carry

        lax.fori_loop(0, CHUNK, zrow, 0)
        for k in range(9):
            pltpu.sync_copy(d0, acc.at[pl.ds(z0 + CHUNK * k, CHUNK)])
        pltpu.sync_copy(d0.at[pl.ds(0, 56)], acc.at[pl.ds(z0 + 576, 56)])
        plsc.subcore_barrier()

        def wait_data(r):
            pltpu.make_async_copy(feat.at[srcv.at[0]], bufs[r], sems[r]).wait()

        def wait_idx(dst_v):
            pltpu.make_async_copy(srcp.at[0, pl.ds(0, GRP)], dst_v, semI).wait()

        def gstep(m, carry):
            pltpu.async_copy(srcp.at[w, pl.ds(m * GRP, GRP)], srcv, semI)
            pltpu.async_copy(dstp.at[w, pl.ds(m * GRP, GRP)], dstv, semI)
            wait_idx(srcv)
            wait_idx(dstv)
            pltpu.async_copy(feat.at[srcv.at[0]], d0, sem0)
            for r in range(GRP):
                if r + 1 < GRP:
                    pltpu.async_copy(feat.at[srcv.at[r + 1]], bufs[(r + 1) % 2],
                                     sems[(r + 1) % 2])
                wait_data(r % 2)
                pltpu.sync_copy(bufs[r % 2], acc.at[dstv.at[r]], add=True)
            return carry

        lax.fori_loop(0, NGRP, gstep, 0)

        plsc.subcore_barrier()
        for k in range(9):
            pltpu.sync_copy(acc.at[pl.ds(z0 + CHUNK * k, CHUNK)], d0)
            pltpu.sync_copy(d0, sum_out.at[c, pl.ds(z0 + CHUNK * k, CHUNK)])
        pltpu.sync_copy(acc.at[pl.ds(z0 + 576, 56)], d0.at[pl.ds(0, 56)])
        pltpu.sync_copy(d0.at[pl.ds(0, 56)], sum_out.at[c, pl.ds(z0 + 576, 56)])

    return pl.kernel(
        body,
        out_type=out_type[0],
        mesh=_SC_MESH,
        scratch_types=scratch,
        name="sage_agg",
    )


_agg = _make_agg(False)


def _gather_body(feat, idxp, out, idxv, buf, sem):
    c = lax.axis_index("c")
    s = lax.axis_index("s")
    w = c * NS + s
    pltpu.sync_copy(idxp.at[w], idxv)
    for t in range(2):
        pltpu.async_copy(feat.at[idxv.at[t]], buf, sem).wait()
        pltpu.sync_copy(buf, out.at[pl.ds(w * TROWS + t * 80, 80)])


_gather = pl.kernel(
    _gather_body,
    out_type=jax.ShapeDtypeStruct((NTP, D), jnp.float32),
    mesh=_SC_MESH,
    scratch_types=[
        pltpu.VMEM((2, 80), jnp.int32),
        pltpu.VMEM((80, D), jnp.float32),
        pltpu.SemaphoreType.DMA,
    ],
    name="sage_gather_xt",
)


RB = 1000


def _layer_body(sums_ref, deg_ref, x_ref, wl_ref, bl_ref, wr_ref, g_ref, b_ref, o_ref):
    su = sums_ref[...]
    deg = deg_ref[...]
    s = su[0] + su[1]
    agg = s * (1.0 / jnp.maximum(deg, 1.0))
    x = x_ref[...]
    hp = jax.lax.dot(agg, wl_ref[...], precision=lax.Precision.HIGHEST,
                     preferred_element_type=jnp.float32)
    hp = hp + jax.lax.dot(x, wr_ref[...], precision=lax.Precision.HIGHEST,
                          preferred_element_type=jnp.float32)
    hp = hp + bl_ref[0][None, :]
    scale = g_ref[0] * jax.lax.rsqrt(jnp.float32(1.0 + EPS))
    h = hp * scale[None, :] + b_ref[0][None, :]
    o_ref[...] = jnp.maximum(h, 0.0)


_layer = pl.pallas_call(
    _layer_body,
    grid=(N // RB,),
    in_specs=[
        pl.BlockSpec((NC, RB, D), lambda i: (0, i, 0)),
        pl.BlockSpec((RB, 1), lambda i: (i, 0)),
        pl.BlockSpec((RB, D), lambda i: (i, 0)),
        pl.BlockSpec((D, D), lambda i: (0, 0)),
        pl.BlockSpec((1, D), lambda i: (0, 0)),
        pl.BlockSpec((D, D), lambda i: (0, 0)),
        pl.BlockSpec((1, D), lambda i: (0, 0)),
        pl.BlockSpec((1, D), lambda i: (0, 0)),
    ],
    out_specs=pl.BlockSpec((RB, D), lambda i: (i, 0)),
    out_shape=jax.ShapeDtypeStruct((N, D), jnp.float32),
)


def _head_body(xt_ref, ci_ref, w_ref, b_ref, o_ref):
    xt = xt_ref[...][:NT, :]
    ci = ci_ref[...]
    colsum = jnp.sum(ci, axis=0)[:, None]
    cf = jax.lax.dot_general(ci, xt, (((0,), (0,)), ((), ())),
                             precision=lax.Precision.HIGHEST,
                             preferred_element_type=jnp.float32)
    cf = cf / colsum
    rowmax = jnp.max(ci, axis=1, keepdims=True)
    colidx = lax.broadcasted_iota(jnp.int32, (NT, C), 1)
    amax = jnp.min(jnp.where(ci == rowmax, colidx, C), axis=1, keepdims=True)
    oh = (colidx == amax).astype(jnp.float32)
    x1 = jax.lax.dot(oh, cf, precision=lax.Precision.HIGHEST,
                     preferred_element_type=jnp.float32)
    w = w_ref[...]
    wa = w[:D, :]
    wb = w[D:, :]
    b = b_ref[0][None, :]
    top = (jax.lax.dot(xt, wa, precision=lax.Precision.HIGHEST,
                       preferred_element_type=jnp.float32)
           + jax.lax.dot(x1, wb, precision=lax.Precision.HIGHEST,
                         preferred_element_type=jnp.float32) + b)
    bot = (jax.lax.dot(x1, wa, precision=lax.Precision.HIGHEST,
                       preferred_element_type=jnp.float32)
           + jax.lax.dot(xt, wb, precision=lax.Precision.HIGHEST,
                         preferred_element_type=jnp.float32) + b)
    o_ref[...] = jnp.concatenate([top, bot], axis=0)


_head = pl.pallas_call(
    _head_body,
    out_shape=jax.ShapeDtypeStruct((2 * NT, 4 * 64), jnp.float32),
)


def kernel(x, edge_index, cluster_index, train_mask, conv0_Wl, conv0_bl, conv0_Wr,
           conv1_Wl, conv1_bl, conv1_Wr, bn0_gamma, bn0_beta, bn1_gamma, bn1_beta,
           fc1_W, fc1_b):
    src = edge_index[0].astype(jnp.int32)
    dst = edge_index[1].astype(jnp.int32)
    pad = EP - E
    srcp = jnp.concatenate([src, jnp.zeros((pad,), jnp.int32)]).reshape(NW, NCH, CHUNK)
    dstp = jnp.concatenate([dst, jnp.full((pad,), N, jnp.int32)]).reshape(NW, NCH, CHUNK)

    sums0 = _agg(x, srcp, dstp)
    deg_col = jax.ops.segment_sum(jnp.ones((E,), jnp.float32), dst,
                                  num_segments=N).reshape(N, 1)
    h0 = _layer(sums0, deg_col, x, conv0_Wl, conv0_bl.reshape(1, D), conv0_Wr,
                bn0_gamma.reshape(1, D), bn0_beta.reshape(1, D))
    sums1 = _agg(h0, srcp, dstp)
    h1 = _layer(sums1, deg_col, h0, conv1_Wl, conv1_bl.reshape(1, D), conv1_Wr,
                bn1_gamma.reshape(1, D), bn1_beta.reshape(1, D))

    tmp = jnp.concatenate([train_mask.astype(jnp.int32),
                           jnp.zeros((NTP - NT,), jnp.int32)]).reshape(NW, 2, 80)
    xt = _gather(h1, tmp)
    return _head(xt, cluster_index, fc1_W, fc1_b.reshape(1, 4 * 64))
